# baseline (speedup 1.0000x reference)
import jax
import jax.numpy as jnp
from jax import lax
from jax.experimental import pallas as pl
from jax.experimental.pallas import tpu as pltpu
from jax.experimental.pallas import tpu_sc as plsc

_TOPK = 8
_B = 256
_K = 17
_HW = 96 * 72
_ROWS = _B * _K
_NC = 2
_NS = 16
_NW = _NC * _NS
_RPW = _ROWS // _NW
_SPW = _B // _NW
_C = 4
_NCHUNK = _RPW // _C
_L = 16
_NVREG = _HW // _L
_UNROLL = 8
_NSTEP = _NVREG // _UNROLL


def _sc_body(o_hbm, t_hbm, w_hbm, out_hbm, o_buf, t_buf, wsq_v, loss_v, sums_v, sems):
    wid = lax.axis_index("s") * _NC + lax.axis_index("c")
    base = wid * _RPW
    idx = lax.iota(jnp.int32, _L)
    z = jnp.zeros((_L,), jnp.float32)

    pltpu.sync_copy(w_hbm.at[pl.ds(wid * _SPW, _SPW), :], wsq_v)
    for s in range(_SPW):
        loss_v[s, pl.ds(0, _L)] = z
        loss_v[s, pl.ds(_L, _L)] = z

    def o_copy(g, slot):
        return pltpu.make_async_copy(
            o_hbm.at[pl.ds(base + g * _C, _C), :], o_buf.at[slot], sems.at[slot])

    def t_copy(g, slot):
        return pltpu.make_async_copy(
            t_hbm.at[pl.ds(base + g * _C, _C), :], t_buf.at[slot], sems.at[2 + slot])

    o_copy(0, 0).start()
    t_copy(0, 0).start()
    o_copy(1, 1).start()
    t_copy(1, 1).start()

    def outer(step, carry):
        for slot in range(2):
            g = step * 2 + slot
            o_copy(g, slot).wait()
            t_copy(g, slot).wait()

            @pl.when(g + 2 < _NCHUNK)
            def _():
                o_copy(g + 2, slot).start()
                t_copy(g + 2, slot).start()

            for r in range(_C):
                def inner(i, accs, _slot=slot, _r=r):
                    news = list(accs)
                    for j in range(_UNROLL):
                        off = (i * _UNROLL + j) * _L
                        ov = o_buf[_slot, _r, pl.ds(off, _L)]
                        tv = t_buf[_slot, _r, pl.ds(off, _L)]
                        d = ov - tv
                        news[j % 4] = news[j % 4] + d * d
                    return tuple(news)

                a0, a1, a2, a3 = lax.fori_loop(0, _NSTEP, inner, (z, z, z, z))
                meanv = jnp.sum((a0 + a1) + (a2 + a3)) * (1.0 / _HW)
                lr = g * _C + r
                s_idx = lr // _K
                k_idx = lr % _K

                @pl.when(k_idx < _L)
                def _():
                    cur = loss_v[s_idx, pl.ds(0, _L)]
                    loss_v[s_idx, pl.ds(0, _L)] = cur + jnp.where(
                        idx == k_idx, meanv, 0.0)

                @pl.when(k_idx == _L)
                def _():
                    cur = loss_v[s_idx, pl.ds(_L, _L)]
                    loss_v[s_idx, pl.ds(_L, _L)] = cur + jnp.where(
                        idx == 0, meanv, 0.0)
        return carry

    lax.fori_loop(0, _NCHUNK // 2, outer, 0)

    total = z
    for s in range(_SPW):
        vals = loss_v[s, pl.ds(0, _L)] * wsq_v[s, pl.ds(0, _L)]
        x17 = (loss_v[s, pl.ds(_L, _L)] * wsq_v[s, pl.ds(_L, _L)])[0]
        srt = plsc.sort_key_val(vals, vals, descending=True)
        if isinstance(srt, (tuple, list)):
            srt = srt[0]
        sum7 = jnp.sum(jnp.where(idx < _TOPK - 1, srt, 0.0))
        s8 = srt[_TOPK - 1]
        total = total + jnp.where(idx == s, sum7 + jnp.maximum(x17, s8), 0.0)
    sums_v[...] = total
    pltpu.sync_copy(sums_v, out_hbm.at[wid])


def kernel(output, target, target_weight):
    o2 = output.reshape(_ROWS, _HW)
    t2 = target.reshape(_ROWS, _HW)
    wsq = (target_weight * target_weight).reshape(_B, _K)
    wsq_pad = jnp.zeros((_B, 2 * _L), jnp.float32).at[:, :_K].set(wsq)
    mesh = plsc.VectorSubcoreMesh(core_axis_name="c", subcore_axis_name="s")
    f = pl.kernel(
        _sc_body,
        out_type=jax.ShapeDtypeStruct((_NW, _L), jnp.float32),
        mesh=mesh,
        compiler_params=pltpu.CompilerParams(needs_layout_passes=False),
        scratch_types=[
            pltpu.VMEM((2, _C, _HW), jnp.float32),
            pltpu.VMEM((2, _C, _HW), jnp.float32),
            pltpu.VMEM((_SPW, 2 * _L), jnp.float32),
            pltpu.VMEM((_SPW, 2 * _L), jnp.float32),
            pltpu.VMEM((_L,), jnp.float32),
            pltpu.SemaphoreType.DMA((4,)),
        ],
    )
    part = f(o2, t2, wsq_pad)
    return jnp.sum(part) / (_TOPK * _B)


# speedup vs baseline: 6.5625x; 6.5625x over previous
import jax
import jax.numpy as jnp
from jax import lax
from jax.experimental import pallas as pl
from jax.experimental.pallas import tpu as pltpu
from jax.experimental.pallas import tpu_sc as plsc

_TOPK = 8
_B = 256
_K = 17
_HW = 96 * 72
_NC = 2
_NS = 16
_L = 16
_CB = _B // _NC
_NG = _CB // _L
_PPT = _HW // _NS
_NSLOT = 3
_CP = _PPT // _NSLOT
_PUNROLL = 4
_NSTEP = _CP // _PUNROLL


def _sc_body(x_hbm, y_hbm, out_hbm, o_buf, t_buf, partial_v, sems):
    c = lax.axis_index("c")
    s = lax.axis_index("s")
    cbase = c * _CB
    pbase = s * _PPT
    z = jnp.zeros((_L,), jnp.float32)

    def o_copy(ki, slot):
        return pltpu.make_async_copy(
            x_hbm.at[ki, pl.ds(pbase + slot * _CP, _CP), pl.ds(cbase, _CB)],
            o_buf.at[slot], sems.at[slot])

    def t_copy(ki, slot):
        return pltpu.make_async_copy(
            y_hbm.at[ki, pl.ds(pbase + slot * _CP, _CP), pl.ds(cbase, _CB)],
            t_buf.at[slot], sems.at[_NSLOT + slot])

    for slot in range(_NSLOT):
        o_copy(0, slot).start()
        t_copy(0, slot).start()

    def outer(ki, carry):
        accs = (z,) * _NG
        for slot in range(_NSLOT):
            o_copy(ki, slot).wait()
            t_copy(ki, slot).wait()

            def inner(i, a, _slot=slot):
                a = list(a)
                for u in range(_PUNROLL):
                    pp = i * _PUNROLL + u
                    for gg in range(_NG):
                        ov = o_buf[_slot, pp, pl.ds(gg * _L, _L)]
                        tv = t_buf[_slot, pp, pl.ds(gg * _L, _L)]
                        d = ov - tv
                        a[gg] = a[gg] + d * d
                return tuple(a)

            accs = lax.fori_loop(0, _NSTEP, inner, accs)

            @pl.when(ki + 1 < _K)
            def _():
                o_copy(ki + 1, slot).start()
                t_copy(ki + 1, slot).start()

        for gg in range(_NG):
            partial_v[ki, pl.ds(gg * _L, _L)] = accs[gg]
        return carry

    lax.fori_loop(0, _K, outer, 0)
    pltpu.sync_copy(partial_v, out_hbm.at[c, s])


def _tc_body(p_ref, w_ref, o_ref):
    x = p_ref[...]
    losses = jnp.sum(x, axis=1) * w_ref[...]
    tot = jnp.zeros((_NC, 1, _CB), jnp.float32)
    for _ in range(_TOPK):
        m = jnp.max(losses, axis=1, keepdims=True)
        tot = tot + m
        eq = losses == m
        taken = jnp.zeros((_NC, 1, _CB), jnp.bool_)
        cols = []
        for k in range(_K):
            ek = eq[:, k:k + 1, :] & (~taken)
            cols.append(jnp.where(ek, -1.0, losses[:, k:k + 1, :]))
            taken = taken | ek
        losses = jnp.concatenate(cols, axis=1)
    o_ref[...] = jnp.full((1, _CB), jnp.sum(tot) * (1.0 / (_TOPK * _B)),
                          jnp.float32)


def kernel(output, target, target_weight):
    x3 = output.transpose(1, 2, 3, 0).reshape(_K, _HW, _B)
    y3 = target.transpose(1, 2, 3, 0).reshape(_K, _HW, _B)
    wsq = ((target_weight * target_weight).reshape(_B, _K).T * (1.0 / _HW))
    wsq = wsq.reshape(_K, _NC, _CB).transpose(1, 0, 2)

    mesh = plsc.VectorSubcoreMesh(core_axis_name="c", subcore_axis_name="s")
    sc = pl.kernel(
        _sc_body,
        out_type=jax.ShapeDtypeStruct((_NC, _NS, _K, _CB), jnp.float32),
        mesh=mesh,
        compiler_params=pltpu.CompilerParams(needs_layout_passes=False),
        scratch_types=[
            pltpu.VMEM((_NSLOT, _CP, _CB), jnp.float32),
            pltpu.VMEM((_NSLOT, _CP, _CB), jnp.float32),
            pltpu.VMEM((_K, _CB), jnp.float32),
            pltpu.SemaphoreType.DMA((2 * _NSLOT,)),
        ],
    )
    partials = sc(x3, y3)

    total = pl.pallas_call(
        _tc_body,
        out_shape=jax.ShapeDtypeStruct((1, _CB), jnp.float32),
    )(partials, wsq)
    return total[0, 0]
